# baseline (speedup 1.0000x reference)
import jax
import jax.numpy as jnp
from jax.experimental import pallas as pl
from jax.experimental.pallas import tpu as pltpu

N = 1048576
BLK = 8192
NBLK = N // BLK
MEAN = (N - 1) / 2.0
VAR_UNB = N * (N + 1) / 12.0


def _dot_body(rp_ref, rt_ref, out_ref, acc_ref, c_ref):
    i = pl.program_id(0)

    @pl.when(i == 0)
    def _():
        acc_ref[0] = 0.0
        c_ref[0] = 0.0

    cp = rp_ref[...] - MEAN
    ct = rt_ref[...] - MEAN
    part = jnp.sum(cp * ct)
    y = part - c_ref[0]
    t = acc_ref[0] + y
    c_ref[0] = (t - acc_ref[0]) - y
    acc_ref[0] = t

    @pl.when(i == NBLK - 1)
    def _():
        cov = acc_ref[0] / N
        out_ref[0, 0] = -(cov / (VAR_UNB + 1e-12))


def kernel(y_pred, y_true):
    rp = jnp.argsort(jnp.argsort(y_pred)).astype(jnp.float32)
    rt = jnp.argsort(jnp.argsort(y_true)).astype(jnp.float32)
    out = pl.pallas_call(
        _dot_body,
        grid=(NBLK,),
        in_specs=[
            pl.BlockSpec((BLK,), lambda i: (i,)),
            pl.BlockSpec((BLK,), lambda i: (i,)),
        ],
        out_specs=pl.BlockSpec(memory_space=pltpu.SMEM),
        out_shape=jax.ShapeDtypeStruct((1, 1), jnp.float32),
        scratch_shapes=[
            pltpu.SMEM((1,), jnp.float32),
            pltpu.SMEM((1,), jnp.float32),
        ],
    )(rp, rt)
    return out[0, 0]
